# baseline (device time: 100186 ns/iter reference)
import jax
import jax.numpy as jnp
from jax import lax
from jax.experimental import pallas as pl
from jax.experimental.pallas import tpu as pltpu

G = 4
C = 4
B = 3


def kernel(x, w_mat):
    k_glob, kc = x.shape
    n_dev = k_glob // kc
    m_per = kc
    _, n = w_mat.shape
    kg = k_glob // G
    nc = n // C
    spg = n_dev // G

    def body(x_ref, w_ref, out_ref, xg_ref, amax_ref, wbuf, wsem,
             xs_sem, xr_sem, as_sem, ar_sem):
        me = lax.axis_index("i")
        my_g = me // spg

        def x_rdma(d, src_slot):
            return pltpu.make_async_remote_copy(
                src_ref=x_ref.at[pl.ds(d * m_per, m_per), :],
                dst_ref=xg_ref.at[:, pl.ds(src_slot * kc, kc)],
                send_sem=xs_sem.at[d],
                recv_sem=xr_sem.at[src_slot],
                device_id=(d,),
                device_id_type=pl.DeviceIdType.MESH,
            )

        def send_all():
            for off in range(1, n_dev):
                d = (me + off) % n_dev
                x_rdma(d, me).start()

        def w_dma(i, slot):
            g, c = divmod(i, C)
            return pltpu.make_async_copy(
                w_ref.at[pl.ds(g * kg, kg), pl.ds(c * nc, nc)],
                wbuf.at[slot],
                wsem.at[slot],
            )

        xg_ref[:, pl.ds(me * kc, kc)] = x_ref[pl.ds(me * m_per, m_per), :]

        @pl.when(my_g == 0)
        def _():
            send_all()

        for i in range(B):
            w_dma(i, i).start()

        for i in range(G * C):
            g, c = divmod(i, C)
            if c == 0:
                for s in range(g * spg, (g + 1) * spg):
                    @pl.when(s != me)
                    def _():
                        x_rdma(me, s).wait_recv()
                if g + 1 < G:
                    @pl.when(my_g == g + 1)
                    def _():
                        send_all()

            slot = i % B
            w_dma(i, slot).wait()
            a_op = xg_ref[:, g * kg:(g + 1) * kg].astype(jnp.bfloat16)
            w_op = wbuf[slot].astype(jnp.bfloat16)
            prod = jnp.dot(a_op, w_op, preferred_element_type=jnp.float32)
            if g == 0:
                out_ref[:, c * nc:(c + 1) * nc] = prod
            else:
                out_ref[:, c * nc:(c + 1) * nc] += prod
            if i + B < G * C:
                w_dma(i + B, slot).start()

        for off in range(1, n_dev):
            d = (me + off) % n_dev
            x_rdma(d, me).wait_send()

        local_amax = jnp.max(jnp.abs(out_ref[:, :]))
        amax_ref[pl.ds(me, 1)] = jnp.full((1, 8, 128), local_amax,
                                          jnp.float32)

        def a_rdma(d, src_slot):
            return pltpu.make_async_remote_copy(
                src_ref=amax_ref.at[me],
                dst_ref=amax_ref.at[src_slot],
                send_sem=as_sem.at[d],
                recv_sem=ar_sem.at[src_slot],
                device_id=(d,),
                device_id_type=pl.DeviceIdType.MESH,
            )

        for off in range(1, n_dev):
            d = (me + off) % n_dev
            a_rdma(d, me).start()
        for off in range(1, n_dev):
            s = (me + off) % n_dev
            a_rdma(me, s).wait_recv()
        for off in range(1, n_dev):
            d = (me + off) % n_dev
            a_rdma(d, me).wait_send()

        g_amax = jnp.max(amax_ref[:, :, :])
        scale = g_amax / 448.0
        y = out_ref[:, :] / scale
        q = jnp.clip(y, -448.0, 448.0).astype(jnp.float8_e4m3fn)
        out_ref[:, :] = q.astype(jnp.float32) * scale

    return pl.pallas_call(
        body,
        out_shape=jax.ShapeDtypeStruct((m_per, n), jnp.float32),
        in_specs=[
            pl.BlockSpec(memory_space=pltpu.VMEM),
            pl.BlockSpec(memory_space=pl.ANY),
        ],
        out_specs=pl.BlockSpec(memory_space=pltpu.VMEM),
        scratch_shapes=[
            pltpu.VMEM((m_per, k_glob), jnp.float32),
            pltpu.VMEM((n_dev, 8, 128), jnp.float32),
            pltpu.VMEM((B, kg, nc), jnp.float32),
            pltpu.SemaphoreType.DMA((B,)),
            pltpu.SemaphoreType.DMA((n_dev,)),
            pltpu.SemaphoreType.DMA((n_dev,)),
            pltpu.SemaphoreType.DMA((n_dev,)),
            pltpu.SemaphoreType.DMA((n_dev,)),
        ],
        compiler_params=pltpu.CompilerParams(
            vmem_limit_bytes=100 * 1024 * 1024,
        ),
    )(x, w_mat)


# device time: 80428 ns/iter; 1.2457x vs baseline; 1.2457x over previous
import jax
import jax.numpy as jnp
from jax import lax
from jax.experimental import pallas as pl
from jax.experimental.pallas import tpu as pltpu

CH = 4
C = 4
B = 6


def kernel(x, w_mat):
    k_glob, kc = x.shape
    n_dev = k_glob // kc
    m_per = kc
    _, n = w_mat.shape
    ks = CH * kc
    nc = n // C
    P = n_dev // CH
    n_slab = P * C

    def body(x_ref, w_ref, out_ref, xg_ref, amax_ref, wbuf, wsem,
             xs_sem, xr_sem, as_sem, ar_sem):
        me = lax.axis_index("i")

        def x_rdma(off):
            d = lax.rem(me + n_dev - off, n_dev)
            return pltpu.make_async_remote_copy(
                src_ref=x_ref.at[pl.ds(d * m_per, m_per), :],
                dst_ref=xg_ref.at[:, pl.ds(off * kc, kc)],
                send_sem=xs_sem.at[off],
                recv_sem=xr_sem.at[off],
                device_id=(d,),
                device_id_type=pl.DeviceIdType.MESH,
            )

        def x_recv(t):
            return pltpu.make_async_remote_copy(
                src_ref=x_ref.at[pl.ds(0, m_per), :],
                dst_ref=xg_ref.at[:, pl.ds(t * kc, kc)],
                send_sem=xs_sem.at[t],
                recv_sem=xr_sem.at[t],
                device_id=(me,),
                device_id_type=pl.DeviceIdType.MESH,
            )

        def w_dmas(i, slot):
            p, c = divmod(i, C)
            copies = []
            for q in range(CH):
                r = lax.rem(me + CH * p + q, n_dev) * kc
                copies.append(pltpu.make_async_copy(
                    w_ref.at[pl.ds(r, kc), pl.ds(c * nc, nc)],
                    wbuf.at[slot, pl.ds(q * kc, kc), :],
                    wsem.at[slot, q],
                ))
            return copies

        xg_ref[:, pl.ds(0, kc)] = x_ref[pl.ds(me * m_per, m_per), :]
        for off in range(1, n_dev):
            x_rdma(off).start()

        for i in range(B):
            for cp in w_dmas(i, i):
                cp.start()

        for i in range(n_slab):
            p, c = divmod(i, C)
            if c == 0:
                for q in range(CH):
                    t = CH * p + q
                    if t > 0:
                        x_recv(t).wait_recv()
            slot = i % B
            for cp in w_dmas(i, slot):
                cp.wait()
            a_op = xg_ref[:, p * ks:(p + 1) * ks]
            prod = jnp.dot(a_op, wbuf[slot],
                           preferred_element_type=jnp.float32)
            if p == 0:
                out_ref[:, c * nc:(c + 1) * nc] = prod
            else:
                out_ref[:, c * nc:(c + 1) * nc] += prod
            if i + B < n_slab:
                for cp in w_dmas(i + B, slot):
                    cp.start()

        for off in range(1, n_dev):
            x_rdma(off).wait_send()

        local_amax = jnp.max(jnp.abs(out_ref[:, :]))
        amax_ref[pl.ds(me, 1)] = jnp.full((1, 8, 128), local_amax,
                                          jnp.float32)

        def a_rdma(d, src_slot):
            return pltpu.make_async_remote_copy(
                src_ref=amax_ref.at[me],
                dst_ref=amax_ref.at[src_slot],
                send_sem=as_sem.at[d],
                recv_sem=ar_sem.at[src_slot],
                device_id=(d,),
                device_id_type=pl.DeviceIdType.MESH,
            )

        for off in range(1, n_dev):
            d = lax.rem(me + off, n_dev)
            a_rdma(d, me).start()
        for off in range(1, n_dev):
            s = lax.rem(me + off, n_dev)
            a_rdma(me, s).wait_recv()
        for off in range(1, n_dev):
            d = lax.rem(me + off, n_dev)
            a_rdma(d, me).wait_send()

        g_amax = jnp.max(amax_ref[:, :, :])
        scale = g_amax / 448.0
        y = out_ref[:, :] / scale
        q = jnp.clip(y, -448.0, 448.0).astype(jnp.float8_e4m3fn)
        out_ref[:, :] = q.astype(jnp.float32) * scale

    return pl.pallas_call(
        body,
        out_shape=jax.ShapeDtypeStruct((m_per, n), jnp.float32),
        in_specs=[
            pl.BlockSpec(memory_space=pltpu.VMEM),
            pl.BlockSpec(memory_space=pl.ANY),
        ],
        out_specs=pl.BlockSpec(memory_space=pltpu.VMEM),
        scratch_shapes=[
            pltpu.VMEM((m_per, k_glob), jnp.float32),
            pltpu.VMEM((n_dev, 8, 128), jnp.float32),
            pltpu.VMEM((B, ks, nc), jnp.float32),
            pltpu.SemaphoreType.DMA((B, CH)),
            pltpu.SemaphoreType.DMA((n_dev,)),
            pltpu.SemaphoreType.DMA((n_dev,)),
            pltpu.SemaphoreType.DMA((n_dev,)),
            pltpu.SemaphoreType.DMA((n_dev,)),
        ],
        compiler_params=pltpu.CompilerParams(
            vmem_limit_bytes=100 * 1024 * 1024,
        ),
    )(x, w_mat)


# device time: 67406 ns/iter; 1.4863x vs baseline; 1.1932x over previous
import jax
import jax.numpy as jnp
from jax import lax
from jax.experimental import pallas as pl
from jax.experimental.pallas import tpu as pltpu

CH = 4
C = 4
B = 8


def kernel(x, w_mat):
    k_glob, kc = x.shape
    n_dev = k_glob // kc
    m_per = kc
    _, n = w_mat.shape
    ks = CH * kc
    nc = n // C
    P = n_dev // CH
    n_slab = P * C

    def body(x_ref, w_ref, out_ref, xg_ref, amax_ref, wbuf, wsem,
             xs_sem, xr_sem, as_sem, ar_sem):
        me = lax.axis_index("i")

        def x_rdma(off):
            d = lax.rem(me + n_dev - off, n_dev)
            return pltpu.make_async_remote_copy(
                src_ref=x_ref.at[pl.ds(d * m_per, m_per), :],
                dst_ref=xg_ref.at[:, pl.ds(off * kc, kc)],
                send_sem=xs_sem.at[off],
                recv_sem=xr_sem.at[off],
                device_id=(d,),
                device_id_type=pl.DeviceIdType.MESH,
            )

        def x_recv(t):
            return pltpu.make_async_remote_copy(
                src_ref=x_ref.at[pl.ds(0, m_per), :],
                dst_ref=xg_ref.at[:, pl.ds(t * kc, kc)],
                send_sem=xs_sem.at[t],
                recv_sem=xr_sem.at[t],
                device_id=(me,),
                device_id_type=pl.DeviceIdType.MESH,
            )

        def w_dmas(i, slot):
            p, c = divmod(i, C)
            copies = []
            for q in range(CH):
                r = lax.rem(me + CH * p + q, n_dev) * kc
                copies.append(pltpu.make_async_copy(
                    w_ref.at[pl.ds(r, kc), pl.ds(c * nc, nc)],
                    wbuf.at[slot, pl.ds(q * kc, kc), :],
                    wsem.at[slot, q],
                ))
            return copies

        xg_ref[:, pl.ds(0, kc)] = x_ref[pl.ds(me * m_per, m_per), :]
        for i in range(B):
            for cp in w_dmas(i, i):
                cp.start()

        barrier_sem = pltpu.get_barrier_semaphore()
        for off in range(1, n_dev):
            d = lax.rem(me + off, n_dev)
            pl.semaphore_signal(barrier_sem, inc=1, device_id=(d,),
                                device_id_type=pl.DeviceIdType.MESH)
        pl.semaphore_wait(barrier_sem, n_dev - 1)

        for off in range(1, n_dev):
            x_rdma(off).start()

        for i in range(n_slab):
            p, c = divmod(i, C)
            if c == 0:
                for q in range(CH):
                    t = CH * p + q
                    if t > 0:
                        x_recv(t).wait_recv()
            slot = i % B
            for cp in w_dmas(i, slot):
                cp.wait()
            a_op = xg_ref[:, p * ks:(p + 1) * ks]
            prod = jnp.dot(a_op, wbuf[slot],
                           preferred_element_type=jnp.float32)
            if p == 0:
                out_ref[:, c * nc:(c + 1) * nc] = prod
            else:
                out_ref[:, c * nc:(c + 1) * nc] += prod
            if i + B < n_slab:
                for cp in w_dmas(i + B, slot):
                    cp.start()

        for off in range(1, n_dev):
            x_rdma(off).wait_send()

        local_amax = jnp.max(jnp.abs(out_ref[:, :]))
        amax_ref[pl.ds(me, 1)] = jnp.full((1, 8, 128), local_amax,
                                          jnp.float32)

        def a_rdma(d, src_slot):
            return pltpu.make_async_remote_copy(
                src_ref=amax_ref.at[me],
                dst_ref=amax_ref.at[src_slot],
                send_sem=as_sem.at[d],
                recv_sem=ar_sem.at[src_slot],
                device_id=(d,),
                device_id_type=pl.DeviceIdType.MESH,
            )

        for off in range(1, n_dev):
            d = lax.rem(me + off, n_dev)
            a_rdma(d, me).start()
        for off in range(1, n_dev):
            s = lax.rem(me + off, n_dev)
            a_rdma(me, s).wait_recv()
        for off in range(1, n_dev):
            d = lax.rem(me + off, n_dev)
            a_rdma(d, me).wait_send()

        g_amax = jnp.max(amax_ref[:, :, :])
        scale = g_amax / 448.0
        y = out_ref[:, :] / scale
        q = jnp.clip(y, -448.0, 448.0).astype(jnp.float8_e4m3fn)
        out_ref[:, :] = q.astype(jnp.float32) * scale

    return pl.pallas_call(
        body,
        out_shape=jax.ShapeDtypeStruct((m_per, n), jnp.float32),
        in_specs=[
            pl.BlockSpec(memory_space=pltpu.VMEM),
            pl.BlockSpec(memory_space=pl.ANY),
        ],
        out_specs=pl.BlockSpec(memory_space=pltpu.VMEM),
        scratch_shapes=[
            pltpu.VMEM((m_per, k_glob), jnp.float32),
            pltpu.VMEM((n_dev, 8, 128), jnp.float32),
            pltpu.VMEM((B, ks, nc), jnp.float32),
            pltpu.SemaphoreType.DMA((B, CH)),
            pltpu.SemaphoreType.DMA((n_dev,)),
            pltpu.SemaphoreType.DMA((n_dev,)),
            pltpu.SemaphoreType.DMA((n_dev,)),
            pltpu.SemaphoreType.DMA((n_dev,)),
        ],
        compiler_params=pltpu.CompilerParams(
            vmem_limit_bytes=100 * 1024 * 1024,
            collective_id=0,
        ),
    )(x, w_mat)


# device time: 64359 ns/iter; 1.5567x vs baseline; 1.0473x over previous
import jax
import jax.numpy as jnp
from jax import lax
from jax.experimental import pallas as pl
from jax.experimental.pallas import tpu as pltpu

CH = 4
C = 4
B = 10


def kernel(x, w_mat):
    k_glob, kc = x.shape
    n_dev = k_glob // kc
    m_per = kc
    _, n = w_mat.shape
    ks = CH * kc
    nc = n // C
    P = n_dev // CH
    n_slab = P * C

    def body(x_ref, w_ref, out_ref, xg_ref, amax_ref, wbuf, wsem,
             xs_sem, xr_sem, as_sem, ar_sem):
        me = lax.axis_index("i")

        def x_rdma(off):
            d = lax.rem(me + n_dev - off, n_dev)
            return pltpu.make_async_remote_copy(
                src_ref=x_ref.at[pl.ds(d * m_per, m_per), :],
                dst_ref=xg_ref.at[:, pl.ds(off * kc, kc)],
                send_sem=xs_sem.at[off],
                recv_sem=xr_sem.at[off],
                device_id=(d,),
                device_id_type=pl.DeviceIdType.MESH,
            )

        def x_recv(t):
            return pltpu.make_async_remote_copy(
                src_ref=x_ref.at[pl.ds(0, m_per), :],
                dst_ref=xg_ref.at[:, pl.ds(t * kc, kc)],
                send_sem=xs_sem.at[t],
                recv_sem=xr_sem.at[t],
                device_id=(me,),
                device_id_type=pl.DeviceIdType.MESH,
            )

        def w_dmas(i, slot):
            p, c = divmod(i, C)
            copies = []
            for q in range(CH):
                r = lax.rem(me + CH * p + q, n_dev) * kc
                copies.append(pltpu.make_async_copy(
                    w_ref.at[pl.ds(r, kc), pl.ds(c * nc, nc)],
                    wbuf.at[slot, pl.ds(q * kc, kc), :],
                    wsem.at[slot, q],
                ))
            return copies

        xg_ref[:, pl.ds(0, kc)] = x_ref[pl.ds(me * m_per, m_per), :]
        for i in range(B):
            for cp in w_dmas(i, i):
                cp.start()

        barrier_sem = pltpu.get_barrier_semaphore()
        for off in range(1, n_dev):
            d = lax.rem(me + off, n_dev)
            pl.semaphore_signal(barrier_sem, inc=1, device_id=(d,),
                                device_id_type=pl.DeviceIdType.MESH)
        pl.semaphore_wait(barrier_sem, n_dev - 1)

        for off in range(1, n_dev):
            x_rdma(off).start()

        chunk_amax = []
        for i in range(n_slab):
            p, c = divmod(i, C)
            if c == 0:
                for q in range(CH):
                    t = CH * p + q
                    if t > 0:
                        x_recv(t).wait_recv()
            slot = i % B
            for cp in w_dmas(i, slot):
                cp.wait()
            a_op = xg_ref[:, p * ks:(p + 1) * ks]
            prod = jnp.dot(a_op, wbuf[slot],
                           preferred_element_type=jnp.float32)
            if p == 0:
                out_ref[:, c * nc:(c + 1) * nc] = prod
            else:
                acc = out_ref[:, c * nc:(c + 1) * nc] + prod
                out_ref[:, c * nc:(c + 1) * nc] = acc
                if p == P - 1:
                    chunk_amax.append(jnp.max(jnp.abs(acc)))
            if i + B < n_slab:
                for cp in w_dmas(i + B, slot):
                    cp.start()

        for off in range(1, n_dev):
            x_rdma(off).wait_send()

        local_amax = jnp.maximum(jnp.maximum(chunk_amax[0], chunk_amax[1]),
                                 jnp.maximum(chunk_amax[2], chunk_amax[3]))
        amax_ref[pl.ds(me, 1)] = jnp.full((1, 8, 128), local_amax,
                                          jnp.float32)

        def a_rdma(d, src_slot):
            return pltpu.make_async_remote_copy(
                src_ref=amax_ref.at[me],
                dst_ref=amax_ref.at[src_slot],
                send_sem=as_sem.at[d],
                recv_sem=ar_sem.at[src_slot],
                device_id=(d,),
                device_id_type=pl.DeviceIdType.MESH,
            )

        for off in range(1, n_dev):
            d = lax.rem(me + off, n_dev)
            a_rdma(d, me).start()
        for off in range(1, n_dev):
            s = lax.rem(me + off, n_dev)
            a_rdma(me, s).wait_recv()
        for off in range(1, n_dev):
            d = lax.rem(me + off, n_dev)
            a_rdma(d, me).wait_send()

        g_amax = jnp.max(amax_ref[:, :, :])
        scale = g_amax / 448.0
        inv = 448.0 / g_amax
        q = (out_ref[:, :] * inv).astype(jnp.float8_e4m3fn)
        out_ref[:, :] = q.astype(jnp.float32) * scale

    return pl.pallas_call(
        body,
        out_shape=jax.ShapeDtypeStruct((m_per, n), jnp.float32),
        in_specs=[
            pl.BlockSpec(memory_space=pltpu.VMEM),
            pl.BlockSpec(memory_space=pl.ANY),
        ],
        out_specs=pl.BlockSpec(memory_space=pltpu.VMEM),
        scratch_shapes=[
            pltpu.VMEM((m_per, k_glob), jnp.float32),
            pltpu.VMEM((n_dev, 8, 128), jnp.float32),
            pltpu.VMEM((B, ks, nc), jnp.float32),
            pltpu.SemaphoreType.DMA((B, CH)),
            pltpu.SemaphoreType.DMA((n_dev,)),
            pltpu.SemaphoreType.DMA((n_dev,)),
            pltpu.SemaphoreType.DMA((n_dev,)),
            pltpu.SemaphoreType.DMA((n_dev,)),
        ],
        compiler_params=pltpu.CompilerParams(
            vmem_limit_bytes=100 * 1024 * 1024,
            collective_id=0,
        ),
    )(x, w_mat)
